# Initial kernel scaffold; baseline (speedup 1.0000x reference)
import jax
import jax.numpy as jnp
from jax.experimental import pallas as pl


def kernel(x, spa_emb_weight):
    raise NotImplementedError("write your pallas kernel here")


# speedup vs baseline: 1.0953x; 1.0953x over previous
import functools

import jax
import jax.numpy as jnp
from jax import lax
from jax.experimental import pallas as pl
from jax.experimental.pallas import tpu as pltpu
from jax.experimental.pallas import tpu_sc as plsc

_B = 16384 * 50
_D = 32
_NW = 32
_BPW = _B // _NW
_C = 1024
_NCHUNK = _BPW // _C


def _gather_body(idx_hbm, table_hbm, out_hbm, idx_v, rows_v, sem):
    wid = lax.axis_index("s") * 2 + lax.axis_index("c")
    base = wid * _BPW

    def chunk(i, carry):
        off = base + i * _C
        pltpu.sync_copy(idx_hbm.at[pl.ds(off, _C)], idx_v)
        pltpu.async_copy(table_hbm.at[idx_v], rows_v, sem).wait()
        pltpu.sync_copy(rows_v, out_hbm.at[pl.ds(off, _C)])
        return carry

    lax.fori_loop(0, _NCHUNK, chunk, 0)


@jax.jit
def kernel(x, spa_emb_weight):
    idx = x.reshape(-1).astype(jnp.int32)
    mesh = plsc.VectorSubcoreMesh(core_axis_name="c", subcore_axis_name="s")
    out = pl.kernel(
        _gather_body,
        out_type=jax.ShapeDtypeStruct((_B, _D), jnp.float32),
        mesh=mesh,
        scratch_types=[
            pltpu.VMEM((_C,), jnp.int32),
            pltpu.VMEM((_C, _D), jnp.float32),
            pltpu.SemaphoreType.DMA,
        ],
        compiler_params=pltpu.CompilerParams(use_tc_tiling_on_sc=False),
    )(idx, spa_emb_weight)
    return out.reshape(x.shape + (_D,))


# speedup vs baseline: 1.1026x; 1.0067x over previous
import functools

import jax
import jax.numpy as jnp
from jax import lax
from jax.experimental import pallas as pl
from jax.experimental.pallas import tpu as pltpu
from jax.experimental.pallas import tpu_sc as plsc

_B = 16384 * 50
_D = 32
_NW = 32
_BPW = _B // _NW
_C = 512
_NCHUNK = _BPW // _C
_NB = 6


def _gather_body(idx_hbm, table_hbm, out_hbm, idx_v, rows_v, sems):
    wid = lax.axis_index("s") * 2 + lax.axis_index("c")
    base = wid * _BPW

    def fire(j, slot):
        pltpu.sync_copy(idx_hbm.at[pl.ds(base + j * _C, _C)], idx_v.at[slot])
        pltpu.async_copy(table_hbm.at[idx_v.at[slot]], rows_v.at[slot],
                         sems.at[slot])

    for b in range(_NB - 1):
        fire(b, b)

    def chunk(i, carry):
        j = i + _NB - 1

        @pl.when(j < _NCHUNK)
        def _():
            fire(j, lax.rem(j, _NB))

        s = lax.rem(i, _NB)
        pltpu.make_async_copy(table_hbm.at[idx_v.at[s]], rows_v.at[s],
                              sems.at[s]).wait()
        pltpu.sync_copy(rows_v.at[s], out_hbm.at[pl.ds(base + i * _C, _C)])
        return carry

    lax.fori_loop(0, _NCHUNK, chunk, 0)


@jax.jit
def kernel(x, spa_emb_weight):
    idx = x.reshape(-1).astype(jnp.int32)
    mesh = plsc.VectorSubcoreMesh(core_axis_name="c", subcore_axis_name="s")
    out = pl.kernel(
        _gather_body,
        out_type=jax.ShapeDtypeStruct((_B, _D), jnp.float32),
        mesh=mesh,
        scratch_types=[
            pltpu.VMEM((_NB, _C), jnp.int32),
            pltpu.VMEM((_NB, _C, _D), jnp.float32),
            pltpu.SemaphoreType.DMA((_NB,)),
        ],
        compiler_params=pltpu.CompilerParams(use_tc_tiling_on_sc=False),
    )(idx, spa_emb_weight)
    return out.reshape(x.shape + (_D,))


# speedup vs baseline: 1.6228x; 1.4718x over previous
import jax
import jax.numpy as jnp
from jax import lax
from jax.experimental import pallas as pl
from jax.experimental.pallas import tpu as pltpu
from jax.experimental.pallas import tpu_sc as plsc

_NB = 16384
_S = 50
_D = 32
_T = 128
_NT = _NB // _T
_NW = 32
_TPW = _NT // _NW


def _body(x_hbm, tab_hbm, out_hbm, xblk, idxs_v, rows_v, tblk, sem_g, sem_o):
    wid = lax.axis_index("s") * 2 + lax.axis_index("c")
    iota = lax.iota(jnp.int32, 16)
    rowsel = [g * 16 + iota for g in range(8)]

    def extract(s, sl):
        colsel = jnp.full((16,), s, dtype=jnp.int32)
        for g in range(8):
            v = plsc.load_gather(xblk, [rowsel[g], colsel])
            idxs_v[sl, pl.ds(g * 16, 16)] = v

    def fire_gather(sl):
        pltpu.async_copy(tab_hbm.at[idxs_v.at[sl]], rows_v.at[sl],
                         sem_g.at[sl])

    def wait_gather(sl):
        pltpu.make_async_copy(tab_hbm.at[idxs_v.at[sl]], rows_v.at[sl],
                              sem_g.at[sl]).wait()

    def wait_out(sl):
        pltpu.make_async_copy(tblk.at[0], out_hbm.at[0, :, 0],
                              sem_o.at[sl]).wait()

    def t_body(lt, carry):
        t = wid * _TPW + lt
        pltpu.sync_copy(x_hbm.at[pl.ds(t * _T, _T), :], xblk)
        extract(0, 0)
        fire_gather(0)

        def s_body(s, c2):
            sl = lax.rem(s, 2)
            sl1 = lax.rem(s + 1, 2)

            @pl.when(s + 1 < _S)
            def _():
                extract(s + 1, sl1)
                fire_gather(sl1)

            wait_gather(sl)

            @pl.when(jnp.logical_or(s >= 2, lt > 0))
            def _():
                wait_out(sl)

            rows_sl = rows_v.at[sl]
            for u in range(4):
                for ci in range(8):
                    colsel = jnp.full((16,), u * 8 + ci, dtype=jnp.int32)
                    for g in range(8):
                        v = plsc.load_gather(rows_sl, [rowsel[g], colsel])
                        tblk[sl, u, ci, pl.ds(g * 16, 16)] = v

            pltpu.async_copy(tblk.at[sl], out_hbm.at[s, :, t], sem_o.at[sl])
            return c2

        lax.fori_loop(0, _S, s_body, 0)
        return carry

    lax.fori_loop(0, _TPW, t_body, 0)
    wait_out(0)
    wait_out(1)


@jax.jit
def kernel(x, spa_emb_weight):
    mesh = plsc.VectorSubcoreMesh(core_axis_name="c", subcore_axis_name="s")
    out6 = pl.kernel(
        _body,
        out_type=jax.ShapeDtypeStruct((_S, _D // 8, _NT, 8, _T), jnp.float32),
        mesh=mesh,
        scratch_types=[
            pltpu.VMEM((_T, _S), jnp.int32),
            pltpu.VMEM((2, _T), jnp.int32),
            pltpu.VMEM((2, _T, _D), jnp.float32),
            pltpu.VMEM((2, _D // 8, 8, _T), jnp.float32),
            pltpu.SemaphoreType.DMA((2,)),
            pltpu.SemaphoreType.DMA((2,)),
        ],
        compiler_params=pltpu.CompilerParams(
            use_tc_tiling_on_sc=False, needs_layout_passes=False),
    )(x, spa_emb_weight)
    return out6.transpose(2, 4, 0, 1, 3).reshape(_NB, _S, _D)
